# baseline (speedup 1.0000x reference)
import functools

import jax
import jax.numpy as jnp
from jax import lax
from jax.experimental import pallas as pl
from jax.experimental.pallas import tpu as pltpu
from jax.experimental.pallas import tpu_sc as plsc

_BATCH = 16384
_DIM = 64
_LANES = 16
_NC = 2
_NS = 16
_NW = _NC * _NS
_B_PER_W = _BATCH // _NW
_GROUPS = _B_PER_W // _LANES

_mesh = plsc.VectorSubcoreMesh(core_axis_name="c", subcore_axis_name="s")


@functools.partial(
    pl.kernel,
    mesh=_mesh,
    out_type=jax.ShapeDtypeStruct((_BATCH * _DIM,), jnp.float32),
    scratch_types=[
        pltpu.VMEM((_B_PER_W,), jnp.int32),
        pltpu.VMEM((_B_PER_W,), jnp.int32),
        pltpu.VMEM((_B_PER_W, 2 * _DIM), jnp.float32),
        pltpu.VMEM((_B_PER_W * _DIM,), jnp.float32),
        pltpu.SemaphoreType.DMA,
    ],
    compiler_params=pltpu.CompilerParams(needs_layout_passes=False),
)
def _embed_gather(idx_hbm, table_hbm, out_hbm, pair_v, half_v, pairs_v, out_v, sem):
    wid = lax.axis_index("s") * _NC + lax.axis_index("c")
    base = wid * _B_PER_W
    pltpu.sync_copy(idx_hbm.at[pl.ds(base, _B_PER_W)], pair_v)
    for i in range(_GROUPS):
        sl = pl.ds(i * _LANES, _LANES)
        row = pair_v[sl] - 1
        pair_v[sl] = row >> 1
        half_v[sl] = (row & 1) << 6
    pltpu.async_copy(table_hbm.at[pair_v], pairs_v, sem).wait()

    lane = jax.lax.iota(jnp.int32, _LANES)

    def select_group(g, _):
        row16 = g * _LANES + lane
        half16 = half_v[pl.ds(g * _LANES, _LANES)]
        pos16 = row16 * _DIM
        for j in range(_DIM):
            x = plsc.load_gather(pairs_v, [row16, half16 + j])
            plsc.store_scatter(out_v, [pos16 + j], x)
        return _

    jax.lax.fori_loop(0, _GROUPS, select_group, None)
    pltpu.sync_copy(out_v, out_hbm.at[pl.ds(base * _DIM, _B_PER_W * _DIM)])


def kernel(action, embedding):
    flat = _embed_gather(
        action.astype(jnp.int32),
        embedding.reshape(500000, 2 * _DIM),
    )
    return flat.reshape(_BATCH, _DIM)


# speedup vs baseline: 3.3780x; 3.3780x over previous
import functools

import jax
import jax.numpy as jnp
from jax import lax
from jax.experimental import pallas as pl
from jax.experimental.pallas import tpu as pltpu
from jax.experimental.pallas import tpu_sc as plsc

_ROWS = 1000000
_BATCH = 16384
_DIM = 64
_LANES = 16
_NC = 2
_NS = 16
_NW = _NC * _NS
_B_PER_W = _BATCH // _NW
_GSZ = 4
_NG = _B_PER_W // _GSZ

_mesh = plsc.VectorSubcoreMesh(core_axis_name="c", subcore_axis_name="s")


def _embed_gather_body(idx_hbm, table_hbm, out_hbm, idx_v, cols_v, out_v, sem):
    wid = lax.axis_index("s") * _NC + lax.axis_index("c")
    base = wid * _B_PER_W
    pltpu.sync_copy(idx_hbm.at[pl.ds(base, _B_PER_W)], idx_v.at[pl.ds(0, _B_PER_W)])

    lane = lax.iota(jnp.int32, _LANES)
    f1_vecs = [(16 * k + lane) >> 3 for k in range(4)]
    f2_vecs = [(16 * k + lane) & 7 for k in range(4)]

    def fire(g, buf):
        rt = (idx_v[pl.ds(g * _GSZ, _LANES)] - 1) >> 7
        for j in range(_GSZ):
            pltpu.async_copy(
                table_hbm.at[:, :, pl.ds(pl.multiple_of(rt[j] * 128, 128), 128)],
                cols_v.at[buf * _GSZ + j],
                sem,
            )

    def extract(g, buf):
        w = (idx_v[pl.ds(g * _GSZ, _LANES)] - 1) & 127
        for j in range(_GSZ):
            pltpu.make_async_copy(
                table_hbm.at[:, :, pl.ds(0, 128)],
                cols_v.at[buf * _GSZ + j],
                sem,
            ).wait()
        for j in range(_GSZ):
            slot = jnp.full((_LANES,), buf * _GSZ + j, jnp.int32)
            wj = jnp.full((_LANES,), w[j], jnp.int32)
            pos = jnp.full((_LANES,), g * _GSZ + j, jnp.int32)
            for k in range(4):
                x = plsc.load_gather(cols_v, [slot, f1_vecs[k], f2_vecs[k], wj])
                plsc.store_scatter(out_v, [f1_vecs[k], f2_vecs[k], pos], x)

    fire(0, 0)

    def pipelined(g, _):
        fire(g, g & 1)
        extract(g - 1, (g - 1) & 1)
        return _

    jax.lax.fori_loop(1, _NG, pipelined, None)
    extract(_NG - 1, (_NG - 1) & 1)

    for f1 in range(8):
        pltpu.sync_copy(out_v.at[f1], out_hbm.at[f1, :, pl.ds(base, _B_PER_W)])


def _make_embed_gather(interpret=False):
    return functools.partial(
        pl.kernel,
        mesh=_mesh,
        out_type=jax.ShapeDtypeStruct((8, 8, _BATCH), jnp.float32),
        scratch_types=[
            pltpu.VMEM((_B_PER_W + _LANES - _GSZ,), jnp.int32),
            pltpu.VMEM((2 * _GSZ, 8, 8, 128), jnp.float32),
            pltpu.VMEM((8, 8, _B_PER_W), jnp.float32),
            pltpu.SemaphoreType.DMA,
        ],
        compiler_params=pltpu.CompilerParams(needs_layout_passes=False),
        interpret=interpret,
    )(_embed_gather_body)


_embed_gather = _make_embed_gather()


def kernel(action, embedding):
    table3 = embedding.T.reshape(8, 8, _ROWS)
    out3 = _embed_gather(action.astype(jnp.int32), table3)
    return out3.reshape(_DIM, _BATCH).T
